# baseline (speedup 1.0000x reference)
import functools

import jax
import jax.numpy as jnp
from jax import lax
from jax.experimental import pallas as pl
from jax.experimental.pallas import tpu as pltpu
from jax.experimental.pallas import tpu_sc as plsc

_DIM = 128
_MAX_W = 32
_B = 64
_S = 1024
_T = _B * _S
_TSC = 32768
_TTC = _T - _TSC
_NW = 32
_TPW = _TSC // _NW
_RPD = 128
_NDMA = _TPW // _RPD
_NBUF = 6
_AHEAD = 5
_TCB = 256


def _sc_body(pos_hbm, mask_hbm, table_hbm, out_hbm, pos_v, mask_v, idx_v,
             rows_v, table_sh, gsem, wsem):
    sid = lax.axis_index("s")
    wid = sid * 2 + lax.axis_index("c")
    base = wid * _TPW

    @pl.when(sid == 0)
    def _():
        pltpu.sync_copy(table_hbm, table_sh)

    pltpu.sync_copy(pos_hbm.at[pl.ds(base, _TPW)], pos_v)
    pltpu.sync_copy(mask_hbm.at[pl.ds(base, _TPW)], mask_v)

    const_row = jnp.full((16,), 1024, jnp.int32)

    def idx_body(i, carry):
        t = i * 16
        pv = pos_v[pl.ds(t, 16)]
        hv = pv & 0xFFFF
        wv = lax.shift_right_logical(pv, 16)
        mv = mask_v[pl.ds(t, 16)]
        flat = hv * _MAX_W + wv
        idx_v[i // 8, pl.ds((i % 8) * 16, 16)] = jnp.where(mv != 0, flat,
                                                           const_row)
        return carry

    lax.fori_loop(0, _TPW // 16, idx_body, 0)
    plsc.subcore_barrier()

    gathers = [None] * _NDMA
    writes = [None] * _NDMA

    def start_gather(j):
        c = pltpu.make_async_copy(table_sh.at[idx_v.at[j]],
                                  rows_v.at[j % _NBUF], gsem)
        c.start()
        return c

    for j in range(min(_AHEAD, _NDMA)):
        gathers[j] = start_gather(j)
    for j in range(_NDMA):
        g = j + _AHEAD
        if g < _NDMA:
            if g >= _NBUF and writes[g - _NBUF] is not None:
                writes[g - _NBUF].wait()
            gathers[g] = start_gather(g)
        gathers[j].wait()
        writes[j] = pltpu.async_copy(
            rows_v.at[j % _NBUF],
            out_hbm.at[pl.ds(base + j * _RPD, _RPD)], wsem)
    for j in range(max(0, _NDMA - _NBUF), _NDMA):
        if writes[j] is not None:
            writes[j].wait()


@functools.partial(jax.jit, static_argnames=())
def _run_sc(pos_packed, mask_flat, table_full):
    fn = pl.kernel(
        _sc_body,
        out_type=jax.ShapeDtypeStruct((_TSC, _DIM), jnp.float32),
        mesh=plsc.VectorSubcoreMesh(core_axis_name="c", subcore_axis_name="s"),
        scratch_types=[
            pltpu.VMEM((_TPW,), jnp.int32),
            pltpu.VMEM((_TPW,), jnp.int32),
            pltpu.VMEM((_NDMA, _RPD), jnp.int32),
            pltpu.VMEM((_NBUF, _RPD, _DIM), jnp.float32),
            pltpu.VMEM_SHARED((1025, _DIM), jnp.float32),
            pltpu.SemaphoreType.DMA,
            pltpu.SemaphoreType.DMA,
        ],
    )
    return fn(pos_packed, mask_flat, table_full)


def _tc_body(pos_ref, mask_ref, tab_ref, out_ref):
    p = pos_ref[0, 0, :]
    m = mask_ref[0, 0, :]
    flat = (p & 0xFFFF) * _MAX_W + lax.shift_right_logical(p, 16)
    onehot = (flat[:, None]
              == lax.broadcasted_iota(jnp.int32, (_TCB, 1024), 1))
    rows = lax.dot_general(
        onehot.astype(jnp.bfloat16), tab_ref[...],
        (((1,), (0,)), ((), ())),
        preferred_element_type=jnp.float32)
    cs = lax.broadcasted_iota(jnp.int32, (_TCB, _DIM), 1) % 2 == 0
    mrow = jnp.where(cs, 1.0, 0.0).astype(jnp.float32)
    out_ref[...] = jnp.where(m[:, None] != 0, rows, mrow)


@functools.partial(jax.jit, static_argnames=())
def _run_tc(pos3, mask3, tab_bf16):
    nblk = _TTC // _TCB
    return pl.pallas_call(
        _tc_body,
        grid=(nblk,),
        in_specs=[
            pl.BlockSpec((1, 1, _TCB), lambda i: (i, 0, 0)),
            pl.BlockSpec((1, 1, _TCB), lambda i: (i, 0, 0)),
            pl.BlockSpec((1024, _DIM), lambda i: (0, 0)),
        ],
        out_specs=pl.BlockSpec((_TCB, _DIM), lambda i: (i, 0)),
        out_shape=jax.ShapeDtypeStruct((_TTC, _DIM), jnp.float32),
    )(pos3, mask3, tab_bf16)


def kernel(pos_idx, pos_idx_mask, table_cos, table_sin):
    comb = jnp.stack([table_cos, table_sin], axis=-1).reshape(1024, _DIM)
    mask_row = jnp.tile(jnp.array([1.0, 0.0], jnp.float32), _DIM // 2)
    table_full = jnp.concatenate([comb, mask_row[None]], axis=0)

    pos_packed = lax.bitcast_convert_type(
        pos_idx.astype(jnp.int16).reshape(_T, 2), jnp.int32)
    mask_flat = pos_idx_mask.astype(jnp.int32).reshape(_T)

    out_sc = _run_sc(pos_packed[:_TSC], mask_flat[:_TSC], table_full)

    nblk = _TTC // _TCB
    pos3 = pos_packed[_TSC:].reshape(nblk, 1, _TCB)
    mask3 = mask_flat[_TSC:].reshape(nblk, 1, _TCB)
    out_tc = _run_tc(pos3, mask3, comb.astype(jnp.bfloat16))

    out = jnp.concatenate([out_sc, out_tc], axis=0)
    return out.reshape(_B, _S, _DIM // 2, 2)


# speedup vs baseline: 1.2448x; 1.2448x over previous
import functools

import jax
import jax.numpy as jnp
from jax import lax
from jax.experimental import pallas as pl
from jax.experimental.pallas import tpu as pltpu
from jax.experimental.pallas import tpu_sc as plsc

_DIM = 128
_MAX_W = 32
_B = 64
_S = 1024
_T = _B * _S
_TSC = 32768
_TTC = _T - _TSC
_NW = 32
_TPW = _TSC // _NW
_RPD = 128
_NDMA = _TPW // _RPD
_NBUF = 6
_AHEAD = 5
_TCB = 512


def _sc_body(pos_hbm, mask_hbm, table_hbm, out_hbm, pos_v, mask_v, idx_v,
             rows_v, table_sh, gsem, wsem):
    sid = lax.axis_index("s")
    wid = sid * 2 + lax.axis_index("c")
    base = wid * _TPW

    @pl.when(sid == 0)
    def _():
        pltpu.sync_copy(table_hbm, table_sh)

    pltpu.sync_copy(pos_hbm.at[pl.ds(base, _TPW)], pos_v)
    pltpu.sync_copy(mask_hbm.at[pl.ds(base, _TPW)], mask_v)

    const_row = jnp.full((16,), 1024, jnp.int32)

    def idx_body(i, carry):
        t = i * 16
        pv = pos_v[pl.ds(t, 16)]
        hv = pv & 0xFFFF
        wv = lax.shift_right_logical(pv, 16)
        mv = mask_v[pl.ds(t, 16)]
        flat = hv * _MAX_W + wv
        idx_v[i // 8, pl.ds((i % 8) * 16, 16)] = jnp.where(mv != 0, flat,
                                                           const_row)
        return carry

    lax.fori_loop(0, _TPW // 16, idx_body, 0)
    plsc.subcore_barrier()

    gathers = [None] * _NDMA
    writes = [None] * _NDMA

    def start_gather(j):
        c = pltpu.make_async_copy(table_sh.at[idx_v.at[j]],
                                  rows_v.at[j % _NBUF], gsem)
        c.start()
        return c

    for j in range(min(_AHEAD, _NDMA)):
        gathers[j] = start_gather(j)
    for j in range(_NDMA):
        g = j + _AHEAD
        if g < _NDMA:
            if g >= _NBUF and writes[g - _NBUF] is not None:
                writes[g - _NBUF].wait()
            gathers[g] = start_gather(g)
        gathers[j].wait()
        writes[j] = pltpu.async_copy(
            rows_v.at[j % _NBUF],
            out_hbm.at[pl.ds(base + j * _RPD, _RPD)], wsem)
    for j in range(max(0, _NDMA - _NBUF), _NDMA):
        if writes[j] is not None:
            writes[j].wait()


@functools.partial(jax.jit, static_argnames=())
def _run_sc(pos_packed, mask_flat, table_full):
    fn = pl.kernel(
        _sc_body,
        out_type=jax.ShapeDtypeStruct((_T, _DIM), jnp.float32),
        mesh=plsc.VectorSubcoreMesh(core_axis_name="c", subcore_axis_name="s"),
        scratch_types=[
            pltpu.VMEM((_TPW,), jnp.int32),
            pltpu.VMEM((_TPW,), jnp.int32),
            pltpu.VMEM((_NDMA, _RPD), jnp.int32),
            pltpu.VMEM((_NBUF, _RPD, _DIM), jnp.float32),
            pltpu.VMEM_SHARED((1025, _DIM), jnp.float32),
            pltpu.SemaphoreType.DMA,
            pltpu.SemaphoreType.DMA,
        ],
    )
    return fn(pos_packed, mask_flat, table_full)


def _tc_body(pos_ref, mask_ref, tab_ref, out_ref):
    p = pos_ref[0, 0, :]
    m = mask_ref[0, 0, :]
    flat = (p & 0xFFFF) * _MAX_W + lax.shift_right_logical(p, 16)
    onehot = (flat[:, None]
              == lax.broadcasted_iota(jnp.int32, (_TCB, 1024), 1))
    rows = lax.dot_general(
        onehot.astype(jnp.bfloat16), tab_ref[...],
        (((1,), (0,)), ((), ())),
        preferred_element_type=jnp.float32)
    cs = lax.broadcasted_iota(jnp.int32, (_TCB, _DIM), 1) % 2 == 0
    mrow = jnp.where(cs, 1.0, 0.0).astype(jnp.float32)
    out_ref[...] = jnp.where(m[:, None] != 0, rows, mrow)


@functools.partial(jax.jit, static_argnames=())
def _run_tc(pos3, mask3, tab_bf16):
    nblk = _TTC // _TCB
    return pl.pallas_call(
        _tc_body,
        grid=(nblk,),
        in_specs=[
            pl.BlockSpec((1, 1, _TCB), lambda i: (i, 0, 0)),
            pl.BlockSpec((1, 1, _TCB), lambda i: (i, 0, 0)),
            pl.BlockSpec((1024, _DIM), lambda i: (0, 0)),
        ],
        out_specs=pl.BlockSpec((_TCB, _DIM), lambda i: (i, 0)),
        out_shape=jax.ShapeDtypeStruct((_TTC, _DIM), jnp.float32),
    )(pos3, mask3, tab_bf16)


def kernel(pos_idx, pos_idx_mask, table_cos, table_sin):
    comb = jnp.stack([table_cos, table_sin], axis=-1).reshape(1024, _DIM)
    mask_row = jnp.tile(jnp.array([1.0, 0.0], jnp.float32), _DIM // 2)
    table_full = jnp.concatenate([comb, mask_row[None]], axis=0)

    pos_packed = lax.bitcast_convert_type(
        pos_idx.astype(jnp.int16).reshape(_T, 2), jnp.int32)
    mask_flat = pos_idx_mask.astype(jnp.int32).reshape(_T)

    out_sc = _run_sc(pos_packed[:_TSC], mask_flat[:_TSC], table_full)

    nblk = _TTC // _TCB
    pos3 = pos_packed[_TSC:].reshape(nblk, 1, _TCB)
    mask3 = mask_flat[_TSC:].reshape(nblk, 1, _TCB)
    out_tc = _run_tc(pos3, mask3, comb.astype(jnp.bfloat16))

    out = lax.dynamic_update_slice(out_sc, out_tc, (_TSC, 0))
    return out.reshape(_B, _S, _DIM // 2, 2)


# speedup vs baseline: 1.2482x; 1.0027x over previous
import functools

import jax
import jax.numpy as jnp
from jax import lax
from jax.experimental import pallas as pl
from jax.experimental.pallas import tpu as pltpu
from jax.experimental.pallas import tpu_sc as plsc

_DIM = 128
_MAX_W = 32
_B = 64
_S = 1024
_T = _B * _S
_TSC = 32768
_TTC = _T - _TSC
_NW = 32
_TPW = _TSC // _NW
_RPD = 128
_NDMA = _TPW // _RPD
_NBUF = 6
_AHEAD = 5
_TCB = 512


def _sc_body(pos_hbm, mask_hbm, table_hbm, out_hbm, pos_v, mask_v, idx_v,
             rows_v, table_sh, gsem, wsem):
    sid = lax.axis_index("s")
    wid = sid * 2 + lax.axis_index("c")
    base = wid * _TPW

    @pl.when(sid == 0)
    def _():
        pltpu.sync_copy(table_hbm, table_sh)

    pltpu.sync_copy(pos_hbm.at[pl.ds(base, _TPW)], pos_v)
    pltpu.sync_copy(mask_hbm.at[pl.ds(base, _TPW)], mask_v)

    const_row = jnp.full((16,), 1024, jnp.int32)

    def idx_body(i, carry):
        t = i * 16
        pv = pos_v[pl.ds(t, 16)]
        hv = pv & 0xFFFF
        wv = lax.shift_right_logical(pv, 16)
        mv = mask_v[pl.ds(t, 16)]
        flat = hv * _MAX_W + wv
        idx_v[i // 8, pl.ds((i % 8) * 16, 16)] = jnp.where(mv != 0, flat,
                                                           const_row)
        return carry

    lax.fori_loop(0, _TPW // 16, idx_body, 0)
    plsc.subcore_barrier()

    gathers = [None] * _NDMA
    writes = [None] * _NDMA

    def start_gather(j):
        c = pltpu.make_async_copy(table_sh.at[idx_v.at[j]],
                                  rows_v.at[j % _NBUF], gsem)
        c.start()
        return c

    for j in range(min(_AHEAD, _NDMA)):
        gathers[j] = start_gather(j)
    for j in range(_NDMA):
        g = j + _AHEAD
        if g < _NDMA:
            if g >= _NBUF and writes[g - _NBUF] is not None:
                writes[g - _NBUF].wait()
            gathers[g] = start_gather(g)
        gathers[j].wait()
        writes[j] = pltpu.async_copy(
            rows_v.at[j % _NBUF],
            out_hbm.at[pl.ds(base + j * _RPD, _RPD)], wsem)
    for j in range(max(0, _NDMA - _NBUF), _NDMA):
        if writes[j] is not None:
            writes[j].wait()


@functools.partial(jax.jit, static_argnames=())
def _run_sc(pos_packed, mask_flat, table_full):
    fn = pl.kernel(
        _sc_body,
        out_type=jax.ShapeDtypeStruct((_T, _DIM), jnp.float32),
        mesh=plsc.VectorSubcoreMesh(core_axis_name="c", subcore_axis_name="s"),
        scratch_types=[
            pltpu.VMEM((_TPW,), jnp.int32),
            pltpu.VMEM((_TPW,), jnp.int32),
            pltpu.VMEM((_NDMA, _RPD), jnp.int32),
            pltpu.VMEM((_NBUF, _RPD, _DIM), jnp.float32),
            pltpu.VMEM_SHARED((1025, _DIM), jnp.float32),
            pltpu.SemaphoreType.DMA,
            pltpu.SemaphoreType.DMA,
        ],
    )
    return fn(pos_packed, mask_flat, table_full)


def _tc_body(pos_ref, mask_ref, tab_ref, prev_ref, out_ref):
    del prev_ref
    p = pos_ref[0, 0, :]
    m = mask_ref[0, 0, :]
    flat = (p & 0xFFFF) * _MAX_W + lax.shift_right_logical(p, 16)
    onehot = (flat[:, None]
              == lax.broadcasted_iota(jnp.int32, (_TCB, 1024), 1))
    rows = lax.dot_general(
        onehot.astype(jnp.bfloat16), tab_ref[...],
        (((1,), (0,)), ((), ())),
        preferred_element_type=jnp.float32)
    cs = lax.broadcasted_iota(jnp.int32, (_TCB, _DIM), 1) % 2 == 0
    mrow = jnp.where(cs, 1.0, 0.0).astype(jnp.float32)
    out_ref[...] = jnp.where(m[:, None] != 0, rows, mrow)


@functools.partial(jax.jit, static_argnames=())
def _run_tc(pos3, mask3, tab_bf16, prev):
    nblk = _TTC // _TCB
    return pl.pallas_call(
        _tc_body,
        grid=(nblk,),
        in_specs=[
            pl.BlockSpec((1, 1, _TCB), lambda i: (i, 0, 0)),
            pl.BlockSpec((1, 1, _TCB), lambda i: (i, 0, 0)),
            pl.BlockSpec((1024, _DIM), lambda i: (0, 0)),
            pl.BlockSpec(memory_space=pl.ANY),
        ],
        out_specs=pl.BlockSpec((_TCB, _DIM), lambda i: (_TSC // _TCB + i, 0)),
        out_shape=jax.ShapeDtypeStruct((_T, _DIM), jnp.float32),
        input_output_aliases={3: 0},
    )(pos3, mask3, tab_bf16, prev)


def kernel(pos_idx, pos_idx_mask, table_cos, table_sin):
    comb = jnp.stack([table_cos, table_sin], axis=-1).reshape(1024, _DIM)
    mask_row = jnp.tile(jnp.array([1.0, 0.0], jnp.float32), _DIM // 2)
    table_full = jnp.concatenate([comb, mask_row[None]], axis=0)

    pos_packed = lax.bitcast_convert_type(
        pos_idx.astype(jnp.int16).reshape(_T, 2), jnp.int32)
    mask_flat = pos_idx_mask.astype(jnp.int32).reshape(_T)

    out_sc = _run_sc(pos_packed[:_TSC], mask_flat[:_TSC], table_full)

    nblk = _TTC // _TCB
    pos3 = pos_packed[_TSC:].reshape(nblk, 1, _TCB)
    mask3 = mask_flat[_TSC:].reshape(nblk, 1, _TCB)
    out = _run_tc(pos3, mask3, comb.astype(jnp.bfloat16), out_sc)
    return out.reshape(_B, _S, _DIM // 2, 2)
